# Initial kernel scaffold; baseline (speedup 1.0000x reference)
import jax
import jax.numpy as jnp
from jax.experimental import pallas as pl


def kernel(x, edge_index, edge_attr, batch, W1l, b1l, W1r, W2l, b2l, W2r, W3l, b3l, W3r, Wlin, blin):
    raise NotImplementedError("write your pallas kernel here")


# speedup vs baseline: 7.7326x; 7.7326x over previous
import functools

import jax
import jax.numpy as jnp
from jax import lax
from jax.experimental import pallas as pl
from jax.experimental.pallas import tpu as pltpu, tpu_sc as plsc

N = 10000
E = 320000
DIN = 128
H = 32
WC = 48
G = 64
C = 2

NW = 32
CE = 128
NCH = 80
EPAD = NW * NCH * CE - E
NP = 10112
RPT = NP // 16
DUMMY = 10016

_f32 = jnp.float32


def _sc_body(width, y_hbm, src_hbm, dst_hbm, z_hbm, part_hbm,
             src_v, dst_v, rows_v, sem, acc):
    del width
    c = lax.axis_index("c")
    s = lax.axis_index("s")
    wid = s * 2 + c
    r0 = s * RPT
    pltpu.sync_copy(z_hbm.at[pl.ds(r0, RPT)], acc.at[pl.ds(r0, RPT)])
    pltpu.sync_copy(src_hbm.at[wid], src_v)
    pltpu.sync_copy(dst_hbm.at[wid], dst_v)
    plsc.subcore_barrier()

    def step(j, carry):
        pltpu.async_copy(y_hbm.at[src_v.at[j]], rows_v, sem).wait()
        pltpu.sync_copy(rows_v, acc.at[dst_v.at[j]], add=True)
        return carry

    lax.fori_loop(0, NCH, step, 0)
    plsc.subcore_barrier()
    pltpu.sync_copy(acc.at[pl.ds(r0, RPT)], part_hbm.at[c, pl.ds(r0, RPT)])


def _make_sc(width):
    mesh = plsc.VectorSubcoreMesh(core_axis_name="c", subcore_axis_name="s")
    out_type = jax.ShapeDtypeStruct((2, NP, width), _f32)
    scratch = [
        pltpu.VMEM((NCH, CE), jnp.int32),
        pltpu.VMEM((NCH, CE), jnp.int32),
        pltpu.VMEM((CE, width), _f32),
        pltpu.SemaphoreType.DMA,
        pltpu.VMEM_SHARED((NP, width), _f32),
    ]
    return pl.kernel(functools.partial(_sc_body, width),
                     out_type=out_type, mesh=mesh, scratch_types=scratch,
                     compiler_params=pltpu.CompilerParams(
                         use_tc_tiling_on_sc=False))


_sc_agg_w = _make_sc(WC)
_sc_agg = _make_sc(H)


def _tc_first_body(x_ref, wl_ref, wr_ref, y_ref, xr_ref):
    xv = x_ref[...]
    dn = (((1,), (1,)), ((), ()))
    y1 = lax.dot_general(xv, wl_ref[...], dn, preferred_element_type=_f32)
    ones = jnp.ones((NP, 1), _f32)
    zeros = jnp.zeros((NP, WC - H - 1), _f32)
    y_ref[...] = jnp.concatenate([y1, ones, zeros], axis=1)
    xr_ref[...] = lax.dot_general(xv, wr_ref[...], dn, preferred_element_type=_f32)


_tc_first = pl.pallas_call(
    _tc_first_body,
    out_shape=[jax.ShapeDtypeStruct((NP, WC), _f32),
               jax.ShapeDtypeStruct((NP, H), _f32)],
)


def _elu(h):
    return jnp.where(h > 0, h, jnp.exp(jnp.minimum(h, 0.0)) - 1.0)


def _tc_mid1_body(part_ref, xr_ref, b_ref, wl_ref, wr_ref,
                  y_ref, xr2_ref, cnt_ref):
    p = part_ref[...]
    a = p[0] + p[1]
    cnt = jnp.maximum(a[:, H:H + 1], 1.0)
    h = _elu(a[:, :H] / cnt + b_ref[...] + xr_ref[...])
    dn = (((1,), (1,)), ((), ()))
    y_ref[...] = lax.dot_general(h, wl_ref[...], dn, preferred_element_type=_f32)
    xr2_ref[...] = lax.dot_general(h, wr_ref[...], dn, preferred_element_type=_f32)
    cnt_ref[...] = cnt


_tc_mid1 = pl.pallas_call(
    _tc_mid1_body,
    out_shape=[jax.ShapeDtypeStruct((NP, H), _f32),
               jax.ShapeDtypeStruct((NP, H), _f32),
               jax.ShapeDtypeStruct((NP, 1), _f32)],
)


def _tc_mid2_body(part_ref, cnt_ref, xr_ref, b_ref, wl_ref, wr_ref,
                  y_ref, xr2_ref):
    p = part_ref[...]
    h = _elu((p[0] + p[1]) / cnt_ref[...] + b_ref[...] + xr_ref[...])
    dn = (((1,), (1,)), ((), ()))
    y_ref[...] = lax.dot_general(h, wl_ref[...], dn, preferred_element_type=_f32)
    xr2_ref[...] = lax.dot_general(h, wr_ref[...], dn, preferred_element_type=_f32)


_tc_mid2 = pl.pallas_call(
    _tc_mid2_body,
    out_shape=[jax.ShapeDtypeStruct((NP, H), _f32),
               jax.ShapeDtypeStruct((NP, H), _f32)],
)


def _tc_last_body(part_ref, cnt_ref, xr_ref, b_ref, batch_ref, wlin_ref,
                  blin_ref, out_ref):
    p = part_ref[...]
    h = _elu((p[0] + p[1]) / cnt_ref[...] + b_ref[...] + xr_ref[...])
    bt = batch_ref[...]
    oh = (bt == lax.broadcasted_iota(jnp.int32, (1, G), 1)).astype(_f32)
    h_aug = jnp.concatenate([h, jnp.ones((NP, 1), _f32)], axis=1)
    psum = lax.dot_general(oh, h_aug, (((0,), (0,)), ((), ())),
                           preferred_element_type=_f32)
    pooled = psum[:, :H] / jnp.maximum(psum[:, H:H + 1], 1.0)
    logits = lax.dot_general(pooled, wlin_ref[...], (((1,), (1,)), ((), ())),
                             preferred_element_type=_f32) + blin_ref[...]
    m = jnp.max(logits, axis=1, keepdims=True)
    lse = m + jnp.log(jnp.sum(jnp.exp(logits - m), axis=1, keepdims=True))
    out_ref[...] = logits - lse


_tc_last = pl.pallas_call(
    _tc_last_body,
    out_shape=jax.ShapeDtypeStruct((G, C), _f32),
)


def kernel(x, edge_index, edge_attr, batch, W1l, b1l, W1r, W2l, b2l, W2r,
           W3l, b3l, W3r, Wlin, blin):
    del edge_attr
    x_pad = jnp.pad(x, ((0, NP - N), (0, 0)))
    src = jnp.concatenate([edge_index[0], jnp.zeros((EPAD,), jnp.int32)])
    dst = jnp.concatenate([edge_index[1], jnp.full((EPAD,), DUMMY, jnp.int32)])
    src_p = src.reshape(NW, NCH, CE)
    dst_p = dst.reshape(NW, NCH, CE)
    zw = jnp.zeros((NP, WC), _f32)
    zh = jnp.zeros((NP, H), _f32)
    batch_p = jnp.pad(batch, (0, NP - N), constant_values=G).reshape(NP, 1)

    y1, xr1 = _tc_first(x_pad, W1l, W1r)
    part1 = _sc_agg_w(y1, src_p, dst_p, zw)
    y2, xr2, cnt = _tc_mid1(part1, xr1, b1l.reshape(1, H), W2l, W2r)
    part2 = _sc_agg(y2, src_p, dst_p, zh)
    y3, xr3 = _tc_mid2(part2, cnt, xr2, b2l.reshape(1, H), W3l, W3r)
    part3 = _sc_agg(y3, src_p, dst_p, zh)
    return _tc_last(part3, cnt, xr3, b3l.reshape(1, H), batch_p, Wlin,
                    blin.reshape(1, C))
